# baseline (speedup 1.0000x reference)
import functools

import jax
import jax.numpy as jnp
import numpy as np
from jax.experimental import pallas as pl
from jax.experimental.pallas import tpu as pltpu

NUM_HEADS = 16
NUM_KV_HEADS = 4
GROUP = NUM_HEADS // NUM_KV_HEADS


def _group_tables_np(seq_len, dim, scale, theta=10000.0):
    inv_freq = 1.0 / (theta ** (np.arange(0, dim, 2, dtype=np.float64) / dim))
    ang = np.outer(np.arange(seq_len, dtype=np.float64), inv_freq)
    cos = np.cos(ang)
    sin = np.sin(ang)
    cos_cat = np.concatenate([cos, cos], axis=-1)
    sin_lo = np.concatenate([-sin, np.zeros_like(sin)], axis=-1)
    sin_hi = np.concatenate([np.zeros_like(sin), sin], axis=-1)
    ones = np.ones((seq_len, dim))
    zeros = np.zeros((seq_len, dim))

    def build(q_pat, k_pat, v_pat):
        return np.concatenate([np.tile(q_pat, (1, GROUP)) * scale,
                               k_pat, v_pat], axis=-1).astype(np.float32)

    return (build(cos_cat, cos_cat, ones),
            build(sin_lo, sin_lo, zeros),
            build(sin_hi, sin_hi, zeros))


def _fused_kernel(x_ref, wq_ref, wo_ref, cosq_ref, sloq_ref, shiq_ref,
                  o_ref, qkv_scr, attn_scr, *, seq, d, tq, gw):
    g = pl.program_id(1)
    half = d // 2
    n_q = seq // tq

    for qi in range(n_q):
        rows = slice(qi * tq, (qi + 1) * tq)
        acc = jnp.dot(x_ref[rows, :], wq_ref[...],
                      preferred_element_type=jnp.float32)
        r_lo = pltpu.roll(acc, shift=gw - half, axis=1)
        r_hi = pltpu.roll(acc, shift=half, axis=1)
        qkv_scr[rows, :] = (acc * cosq_ref[rows, :] + r_lo * sloq_ref[rows, :]
                            + r_hi * shiq_ref[rows, :]).astype(qkv_scr.dtype)

    tqa = tq // 2
    for qi in range(seq // tqa):
        L = (qi + 1) * tqa
        k = qkv_scr[0:L, GROUP * d:GROUP * d + d]
        v = qkv_scr[0:L, GROUP * d + d:GROUP * d + 2 * d]
        row = jax.lax.broadcasted_iota(jnp.int32, (tqa, L), 0)
        col = jax.lax.broadcasted_iota(jnp.int32, (tqa, L), 1)
        keep = col <= row + qi * tqa
        for u in range(GROUP):
            q_u = qkv_scr[qi * tqa:(qi + 1) * tqa, u * d:(u + 1) * d]
            s = jax.lax.dot_general(q_u, k, (((1,), (1,)), ((), ())),
                                    preferred_element_type=jnp.float32)
            s = jnp.where(keep, s, -jnp.inf)
            m = jnp.max(s, axis=-1, keepdims=True)
            p = jnp.exp(s - m)
            inv = pl.reciprocal(jnp.sum(p, axis=-1, keepdims=True),
                                approx=True)
            pv = jnp.dot(p.astype(jnp.bfloat16), v,
                         preferred_element_type=jnp.float32)
            attn_scr[qi * tqa:(qi + 1) * tqa,
                     pl.ds(g * GROUP * d + u * d, d)] = (
                         pv * inv).astype(attn_scr.dtype)

    @pl.when(g == pl.num_programs(1) - 1)
    def _o_proj():
        o_ref[...] = jnp.dot(attn_scr[...], wo_ref[...],
                             preferred_element_type=jnp.float32
                             ).astype(o_ref.dtype)


def kernel(w_qkv, w_o, hidden_states, sequence_mask):
    S, B, Hd = hidden_states.shape
    d = Hd // NUM_HEADS
    scale = 1.0 / (d ** 0.5)
    gw = (GROUP + 2) * d

    x2d = hidden_states.reshape(S, B * Hd).astype(jnp.bfloat16)

    qc, kc = NUM_HEADS * d, NUM_KV_HEADS * d
    wq_re = jnp.concatenate(
        [jnp.concatenate(
            [w_qkv[:, g * GROUP * d:(g + 1) * GROUP * d],
             w_qkv[:, qc + g * d:qc + (g + 1) * d],
             w_qkv[:, qc + kc + g * d:qc + kc + (g + 1) * d]], axis=1)
         for g in range(NUM_KV_HEADS)], axis=1).astype(jnp.bfloat16)
    wo_bf = w_o.astype(jnp.bfloat16)

    tabs_np = _group_tables_np(S, d, scale)
    tabs = tuple(jnp.asarray(t, jnp.bfloat16) for t in tabs_np)

    kern = functools.partial(_fused_kernel, seq=S, d=d, tq=512, gw=gw)
    out = pl.pallas_call(
        kern,
        out_shape=jax.ShapeDtypeStruct((S, B * Hd), jnp.float32),
        grid=(B, NUM_KV_HEADS),
        in_specs=[
            pl.BlockSpec((S, Hd), lambda i, g: (0, i)),
            pl.BlockSpec((Hd, gw), lambda i, g: (0, g)),
            pl.BlockSpec(wo_bf.shape, lambda i, g: (0, 0)),
            pl.BlockSpec((S, gw), lambda i, g: (0, 0)),
            pl.BlockSpec((S, gw), lambda i, g: (0, 0)),
            pl.BlockSpec((S, gw), lambda i, g: (0, 0)),
        ],
        out_specs=pl.BlockSpec((S, Hd), lambda i, g: (0, i)),
        scratch_shapes=[
            pltpu.VMEM((S, gw), jnp.bfloat16),
            pltpu.VMEM((S, NUM_HEADS * d), jnp.bfloat16),
        ],
        compiler_params=pltpu.CompilerParams(
            dimension_semantics=("parallel", "arbitrary"),
            vmem_limit_bytes=67043328),
    )(x2d, wq_re, wo_bf, *tabs)

    return {"hidden_states": out.reshape(S, B, Hd),
            "sequence_mask": sequence_mask}


# speedup vs baseline: 1.0970x; 1.0970x over previous
import functools

import jax
import jax.numpy as jnp
import numpy as np
from jax.experimental import pallas as pl
from jax.experimental.pallas import tpu as pltpu

NUM_HEADS = 16
NUM_KV_HEADS = 4
GROUP = NUM_HEADS // NUM_KV_HEADS


def _group_tables_np(seq_len, dim, scale, theta=10000.0):
    inv_freq = 1.0 / (theta ** (np.arange(0, dim, 2, dtype=np.float64) / dim))
    ang = np.outer(np.arange(seq_len, dtype=np.float64), inv_freq)
    cos = np.cos(ang)
    sin = np.sin(ang)
    cos_cat = np.concatenate([cos, cos], axis=-1)
    sin_lo = np.concatenate([-sin, np.zeros_like(sin)], axis=-1)
    sin_hi = np.concatenate([np.zeros_like(sin), sin], axis=-1)
    ones = np.ones((seq_len, dim))
    zeros = np.zeros((seq_len, dim))

    def build(q_pat, k_pat, v_pat):
        return np.concatenate([np.tile(q_pat, (1, GROUP)) * scale,
                               k_pat, v_pat], axis=-1).astype(np.float32)

    return (build(cos_cat, cos_cat, ones),
            build(sin_lo, sin_lo, zeros),
            build(sin_hi, sin_hi, zeros))


def _fused_kernel(x_ref, wq_ref, wo_ref, cosq_ref, sloq_ref, shiq_ref,
                  o_ref, qkv_scr, attn_scr, *, seq, d, tq, gw):
    g = pl.program_id(1)
    half = d // 2
    n_q = seq // tq

    for qi in range(n_q):
        rows = slice(qi * tq, (qi + 1) * tq)
        acc = jnp.dot(x_ref[rows, :], wq_ref[...],
                      preferred_element_type=jnp.float32)
        r_lo = pltpu.roll(acc, shift=gw - half, axis=1)
        r_hi = pltpu.roll(acc, shift=half, axis=1)
        qkv_scr[rows, :] = (acc * cosq_ref[rows, :] + r_lo * sloq_ref[rows, :]
                            + r_hi * shiq_ref[rows, :]).astype(qkv_scr.dtype)

    tqa = tq
    for qi in range(seq // tqa):
        L = (qi + 1) * tqa
        k = qkv_scr[0:L, GROUP * d:GROUP * d + d]
        v = qkv_scr[0:L, GROUP * d + d:GROUP * d + 2 * d]
        row = jax.lax.broadcasted_iota(jnp.int32, (tqa, L), 0)
        col = jax.lax.broadcasted_iota(jnp.int32, (tqa, L), 1)
        keep = col <= row + qi * tqa
        for u in range(GROUP):
            q_u = qkv_scr[qi * tqa:(qi + 1) * tqa, u * d:(u + 1) * d]
            s = jax.lax.dot_general(q_u, k, (((1,), (1,)), ((), ())),
                                    preferred_element_type=jnp.float32)
            s = jnp.where(keep, s, -jnp.inf)
            m = jnp.max(s, axis=-1, keepdims=True)
            p = jnp.exp(s - m)
            inv = pl.reciprocal(jnp.sum(p, axis=-1, keepdims=True),
                                approx=True)
            pv = jnp.dot(p.astype(jnp.bfloat16), v,
                         preferred_element_type=jnp.float32)
            attn_scr[qi * tqa:(qi + 1) * tqa,
                     pl.ds(g * GROUP * d + u * d, d)] = (
                         pv * inv).astype(attn_scr.dtype)

    @pl.when(g == pl.num_programs(1) - 1)
    def _o_proj():
        o_ref[...] = jnp.dot(attn_scr[...], wo_ref[...],
                             preferred_element_type=jnp.float32
                             ).astype(o_ref.dtype)


def kernel(w_qkv, w_o, hidden_states, sequence_mask):
    S, B, Hd = hidden_states.shape
    d = Hd // NUM_HEADS
    scale = 1.0 / (d ** 0.5)
    gw = (GROUP + 2) * d

    x2d = hidden_states.reshape(S, B * Hd).astype(jnp.bfloat16)

    qc, kc = NUM_HEADS * d, NUM_KV_HEADS * d
    wq_re = jnp.concatenate(
        [jnp.concatenate(
            [w_qkv[:, g * GROUP * d:(g + 1) * GROUP * d],
             w_qkv[:, qc + g * d:qc + (g + 1) * d],
             w_qkv[:, qc + kc + g * d:qc + kc + (g + 1) * d]], axis=1)
         for g in range(NUM_KV_HEADS)], axis=1).astype(jnp.bfloat16)
    wo_bf = w_o.astype(jnp.bfloat16)

    tabs_np = _group_tables_np(S, d, scale)
    tabs = tuple(jnp.asarray(t, jnp.bfloat16) for t in tabs_np)

    kern = functools.partial(_fused_kernel, seq=S, d=d, tq=512, gw=gw)
    out = pl.pallas_call(
        kern,
        out_shape=jax.ShapeDtypeStruct((S, B * Hd), jnp.float32),
        grid=(B, NUM_KV_HEADS),
        in_specs=[
            pl.BlockSpec((S, Hd), lambda i, g: (0, i)),
            pl.BlockSpec((Hd, gw), lambda i, g: (0, g)),
            pl.BlockSpec(wo_bf.shape, lambda i, g: (0, 0)),
            pl.BlockSpec((S, gw), lambda i, g: (0, 0)),
            pl.BlockSpec((S, gw), lambda i, g: (0, 0)),
            pl.BlockSpec((S, gw), lambda i, g: (0, 0)),
        ],
        out_specs=pl.BlockSpec((S, Hd), lambda i, g: (0, i)),
        scratch_shapes=[
            pltpu.VMEM((S, gw), jnp.bfloat16),
            pltpu.VMEM((S, NUM_HEADS * d), jnp.bfloat16),
        ],
        compiler_params=pltpu.CompilerParams(
            dimension_semantics=("parallel", "arbitrary"),
            vmem_limit_bytes=67043328),
    )(x2d, wq_re, wo_bf, *tabs)

    return {"hidden_states": out.reshape(S, B, Hd),
            "sequence_mask": sequence_mask}
